# baseline (speedup 1.0000x reference)
import functools

import jax
import jax.numpy as jnp
from jax import lax
from jax.experimental import pallas as pl

DROP_FRAC = 0.25
INT_MIN = -(2**31)


def _sortable_key(x):
    x = x + 0.0
    b = lax.bitcast_convert_type(x, jnp.int32)
    return b ^ (lax.shift_right_arithmetic(b, 31) & 0x7FFFFFFF)


def _count(pred):
    return jnp.sum(pred.astype(jnp.int32), keepdims=True).reshape(1, 1)


def _suppress_body(k, x_ref, o_ref):
    x = x_ref[0]
    scores = jnp.sum(x, axis=0)
    key = _sortable_key(scores)

    ts = jnp.zeros((1, 1), jnp.int32) + INT_MIN
    for bit in range(31, -1, -1):
        if bit == 31:
            cand = ts ^ INT_MIN
        else:
            cand = ts | (1 << bit)
        cnt = _count(key >= cand)
        ts = jnp.where(cnt >= k, cand, ts)

    gt = key > ts
    eq = key == ts
    need_eq = k - _count(gt)

    hw = (lax.broadcasted_iota(jnp.int32, (8, 128), 0) * 128
          + lax.broadcasted_iota(jnp.int32, (8, 128), 1))
    m = jnp.zeros((1, 1), jnp.int32)
    for bit in range(10, -1, -1):
        cand = m | (1 << bit)
        c = _count(eq & (hw < cand))
        m = jnp.where(c <= need_eq, cand, m)

    zero = gt | (eq & (hw < m))
    mask = jnp.where(zero, 0.0, 1.0).astype(x.dtype)
    o_ref[0] = x * mask[None]


@jax.jit
def kernel(features):
    B, C, H, W = features.shape
    hw = H * W
    assert hw % 128 == 0
    k = int(DROP_FRAC * hw)
    xr = features.reshape(B, C, hw // 128, 128)
    out = pl.pallas_call(
        functools.partial(_suppress_body, k),
        grid=(B,),
        in_specs=[pl.BlockSpec((1, C, hw // 128, 128), lambda b: (b, 0, 0, 0))],
        out_specs=pl.BlockSpec((1, C, hw // 128, 128), lambda b: (b, 0, 0, 0)),
        out_shape=jax.ShapeDtypeStruct((B, C, hw // 128, 128), features.dtype),
    )(xr)
    return out.reshape(B, C, H, W)


# speedup vs baseline: 1.3979x; 1.3979x over previous
import functools

import jax
import jax.numpy as jnp
from jax import lax
from jax.experimental import pallas as pl

DROP_FRAC = 0.25
INT_MIN = -(2**31)


def _sortable_key(x):
    x = x + 0.0
    b = lax.bitcast_convert_type(x, jnp.int32)
    return b ^ (lax.shift_right_arithmetic(b, 31) & 0x7FFFFFFF)


def _count(pred):
    return jnp.sum(pred.astype(jnp.int32), keepdims=True).reshape(1, 1)


def _wrap32(v):
    v &= 0xFFFFFFFF
    return v - 2**32 if v >= 2**31 else v


def _suppress_body(k, x_ref, o_ref):
    x = x_ref[0]
    scores = jnp.sum(x, axis=0)
    key = _sortable_key(scores)

    tu = jnp.zeros((1, 1), jnp.int32)
    for level in range(8):
        s = 28 - 4 * level
        nib = jnp.zeros((1, 1), jnp.int32)
        for j in range(1, 16):
            cand = (tu | _wrap32(j << s)) ^ INT_MIN
            nib = nib + (_count(key >= cand) >= k).astype(jnp.int32)
        tu = tu | (nib * _wrap32(1 << s))
    ts = tu ^ INT_MIN

    gt = key > ts
    eq = key == ts
    need_eq = (k - _count(gt)).astype(jnp.float32)

    e = eq.astype(jnp.float32)
    lane = lax.broadcasted_iota(jnp.int32, (128, 128), 0)
    lane_t = lax.broadcasted_iota(jnp.int32, (128, 128), 1)
    upper = (lane <= lane_t).astype(jnp.float32)
    incl = jax.lax.dot(e, upper, precision=lax.Precision.HIGHEST)
    row_tot = incl[:, 127:128]
    sub = lax.broadcasted_iota(jnp.int32, (8, 8), 0)
    sub_t = lax.broadcasted_iota(jnp.int32, (8, 8), 1)
    strict_lower = (sub > sub_t).astype(jnp.float32)
    offs = jax.lax.dot(strict_lower, row_tot,
                       precision=lax.Precision.HIGHEST)
    rank_incl = incl + offs

    zero = gt | (eq & (rank_incl <= need_eq))
    mask = jnp.where(zero, 0.0, 1.0).astype(x.dtype)
    o_ref[0] = x * mask[None]


@jax.jit
def kernel(features):
    B, C, H, W = features.shape
    hw = H * W
    assert hw % 128 == 0
    k = int(DROP_FRAC * hw)
    xr = features.reshape(B, C, hw // 128, 128)
    out = pl.pallas_call(
        functools.partial(_suppress_body, k),
        grid=(B,),
        in_specs=[pl.BlockSpec((1, C, hw // 128, 128), lambda b: (b, 0, 0, 0))],
        out_specs=pl.BlockSpec((1, C, hw // 128, 128), lambda b: (b, 0, 0, 0)),
        out_shape=jax.ShapeDtypeStruct((B, C, hw // 128, 128), features.dtype),
    )(xr)
    return out.reshape(B, C, H, W)


# speedup vs baseline: 2.8302x; 2.0246x over previous
import functools

import jax
import jax.numpy as jnp
from jax import lax
from jax.experimental import pallas as pl

DROP_FRAC = 0.25
INT_MIN = -(2**31)


def _sortable_key(x):
    x = x + 0.0
    b = lax.bitcast_convert_type(x, jnp.int32)
    return b ^ (lax.shift_right_arithmetic(b, 31) & 0x7FFFFFFF)


def _sum_body(x_ref, o_ref):
    x = x_ref[0]
    o_ref[...] = jnp.sum(x, axis=1).reshape(1, 1, -1)


def _mask_body(k, s_ref, m_ref):
    s = s_ref[...][:, 0, :]
    nb, hw_n = s.shape
    key = _sortable_key(s)

    def count(pred):
        return jnp.sum(pred.astype(jnp.int32), axis=1, keepdims=True)

    ts = jnp.full((nb, 1), INT_MIN, jnp.int32)
    for bit in range(31, -1, -1):
        if bit == 31:
            cand = ts ^ INT_MIN
        else:
            cand = ts | (1 << bit)
        ts = jnp.where(count(key >= cand) >= k, cand, ts)

    gt = key > ts
    eq = key == ts
    need_eq = k - count(gt)

    hw = lax.broadcasted_iota(jnp.int32, (nb, hw_n), 1)
    m = jnp.zeros((nb, 1), jnp.int32)
    for bit in range(10, -1, -1):
        cand = m | (1 << bit)
        c = count(eq & (hw < cand))
        m = jnp.where(c <= need_eq, cand, m)

    zero = gt | (eq & (hw < m))
    mask = jnp.where(zero, 0.0, 1.0)
    m_ref[...] = mask[:, None, :]


def _mul_body(x_ref, m_ref, o_ref):
    o_ref[0] = x_ref[0] * m_ref[0]


@jax.jit
def kernel(features):
    B, C, H, W = features.shape
    hw = H * W
    k = int(DROP_FRAC * hw)
    xt = jnp.transpose(features, (0, 2, 3, 1)).reshape(B, hw, C)

    sums = pl.pallas_call(
        _sum_body,
        grid=(B,),
        in_specs=[pl.BlockSpec((1, hw, C), lambda b: (b, 0, 0))],
        out_specs=pl.BlockSpec((1, 1, hw), lambda b: (b, 0, 0)),
        out_shape=jax.ShapeDtypeStruct((B, 1, hw), jnp.float32),
    )(xt)

    mask_rows = pl.pallas_call(
        functools.partial(_mask_body, k),
        out_shape=jax.ShapeDtypeStruct((B, 1, hw), jnp.float32),
    )(sums)
    mask_col = mask_rows.reshape(B, hw, 1)

    out = pl.pallas_call(
        _mul_body,
        grid=(B,),
        in_specs=[
            pl.BlockSpec((1, hw, C), lambda b: (b, 0, 0)),
            pl.BlockSpec((1, hw, 1), lambda b: (b, 0, 0)),
        ],
        out_specs=pl.BlockSpec((1, hw, C), lambda b: (b, 0, 0)),
        out_shape=jax.ShapeDtypeStruct((B, hw, C), features.dtype),
    )(xt, mask_col)

    return jnp.transpose(out.reshape(B, H, W, C), (0, 3, 1, 2))


# speedup vs baseline: 3.0966x; 1.0941x over previous
import functools

import jax
import jax.numpy as jnp
from jax import lax
from jax.experimental import pallas as pl

DROP_FRAC = 0.25
INT_MIN = -(2**31)


def _sortable_key(x):
    x = x + 0.0
    b = lax.bitcast_convert_type(x, jnp.int32)
    return b ^ (lax.shift_right_arithmetic(b, 31) & 0x7FFFFFFF)


def _sum_body(x_ref, o_ref):
    x = x_ref[0]
    o_ref[...] = jnp.sum(x, axis=1).reshape(1, 1, -1)


def _mask_body(k, s_ref, m_ref):
    s = s_ref[...][:, 0, :]
    nb, hw_n = s.shape
    key = _sortable_key(s)

    ones_col = jnp.ones((hw_n, 1), jnp.float32)

    def count(pred):
        return jax.lax.dot(pred.astype(jnp.float32), ones_col)

    ts = jnp.full((nb, 1), INT_MIN, jnp.int32)
    for bit in range(31, -1, -1):
        if bit == 31:
            cand = ts ^ INT_MIN
        else:
            cand = ts | (1 << bit)
        ts = jnp.where(count(key >= cand) >= k, cand, ts)

    gt = key > ts
    eq = key == ts
    need_eq = k - count(gt)

    hw = lax.broadcasted_iota(jnp.int32, (nb, hw_n), 1)
    m = jnp.zeros((nb, 1), jnp.int32)
    for bit in range(10, -1, -1):
        cand = m | (1 << bit)
        c = count(eq & (hw < cand))
        m = jnp.where(c <= need_eq, cand, m)

    zero = gt | (eq & (hw < m))
    mask = jnp.where(zero, 0.0, 1.0)
    m_ref[...] = mask[:, None, :]


def _mul_body(x_ref, m_ref, o_ref):
    m_col = m_ref[0].reshape(-1, 1)
    o_ref[0] = x_ref[0] * m_col


@jax.jit
def kernel(features):
    B, C, H, W = features.shape
    hw = H * W
    k = int(DROP_FRAC * hw)
    xt = jnp.transpose(features, (0, 2, 3, 1)).reshape(B, hw, C)

    sums = pl.pallas_call(
        _sum_body,
        grid=(B,),
        in_specs=[pl.BlockSpec((1, hw, C), lambda b: (b, 0, 0))],
        out_specs=pl.BlockSpec((1, 1, hw), lambda b: (b, 0, 0)),
        out_shape=jax.ShapeDtypeStruct((B, 1, hw), jnp.float32),
    )(xt)

    mask_rows = pl.pallas_call(
        functools.partial(_mask_body, k),
        out_shape=jax.ShapeDtypeStruct((B, 1, hw), jnp.float32),
    )(sums)

    out = pl.pallas_call(
        _mul_body,
        grid=(B,),
        in_specs=[
            pl.BlockSpec((1, hw, C), lambda b: (b, 0, 0)),
            pl.BlockSpec((1, 1, hw), lambda b: (b, 0, 0)),
        ],
        out_specs=pl.BlockSpec((1, hw, C), lambda b: (b, 0, 0)),
        out_shape=jax.ShapeDtypeStruct((B, hw, C), features.dtype),
    )(xt, mask_rows)

    return jnp.transpose(out.reshape(B, H, W, C), (0, 3, 1, 2))
